# Initial kernel scaffold; baseline (speedup 1.0000x reference)
import jax
import jax.numpy as jnp
from jax.experimental import pallas as pl


def kernel(x, edge_index, edge_attr, params):
    raise NotImplementedError("write your pallas kernel here")


# speedup vs baseline: 4.7697x; 4.7697x over previous
import functools

import jax
import jax.numpy as jnp
from jax import lax
from jax.experimental import pallas as pl
from jax.experimental.pallas import tpu as pltpu
from jax.experimental.pallas import tpu_sc as plsc

F32 = jnp.float32
EPS = 1e-5
NC, NS, LANES = 2, 16, 16
NW = NC * NS
CH = 128
ZR = 160


def _mesh():
    return plsc.VectorSubcoreMesh(core_axis_name="c", subcore_axis_name="s")


_SC_PARAMS = pltpu.CompilerParams(needs_layout_passes=False)


@functools.lru_cache(maxsize=None)
def _make_deg_kernel(n, nch):
    rpt = n // NS

    def body(col_h, ew_h, out_h, col_v, ew_v, vbuf, acc, sem):
        cid = lax.axis_index("c")
        sid = lax.axis_index("s")
        wid = sid * NC + cid
        pltpu.sync_copy(col_h.at[wid], col_v)
        pltpu.sync_copy(ew_h.at[wid], ew_v)

        def zb(e, _):
            vbuf[e, :] = jnp.zeros((LANES,), F32)
            return 0
        lax.fori_loop(0, CH, zb, 0)
        for k in range(rpt // CH):
            pltpu.sync_copy(vbuf, acc.at[pl.ds(sid * rpt + k * CH, CH)])
        plsc.subcore_barrier()

        for c in range(nch):
            def fill(e, _, c=c):
                fi = jnp.full((LANES,), c * CH + e, jnp.int32)
                vbuf[e, :] = plsc.load_gather(ew_v, [fi])
                return 0
            lax.fori_loop(0, CH, fill, 0)
            pltpu.sync_copy(vbuf, acc.at[col_v.at[c]], add=True)
        plsc.subcore_barrier()
        pltpu.sync_copy(acc.at[pl.ds(sid * rpt, rpt)],
                        out_h.at[cid, pl.ds(sid * rpt, rpt)])

    return pl.kernel(
        body,
        out_type=jax.ShapeDtypeStruct((NC, n, LANES), F32),
        mesh=_mesh(),
        compiler_params=_SC_PARAMS,
        scratch_types=[
            pltpu.VMEM((nch, CH), jnp.int32),
            pltpu.VMEM((nch * CH,), F32),
            pltpu.VMEM((CH, LANES), F32),
            pltpu.VMEM_SHARED((n, LANES), F32),
            pltpu.SemaphoreType.DMA,
        ],
    )


@functools.lru_cache(maxsize=None)
def _make_norm_kernel(n, nch):
    def body(row_h, col_h, ew_h, dinv_h, out_h, row_v, col_v, ew_v, dinv_v,
             norm_v):
        cid = lax.axis_index("c")
        sid = lax.axis_index("s")
        wid = sid * NC + cid
        pltpu.sync_copy(row_h.at[wid], row_v)
        pltpu.sync_copy(col_h.at[wid], col_v)
        pltpu.sync_copy(ew_h.at[wid], ew_v)
        pltpu.sync_copy(dinv_h, dinv_v)

        def chunk(c, _):
            def vec(i, _):
                ir = row_v[c, pl.ds(i * LANES, LANES)]
                ic = col_v[c, pl.ds(i * LANES, LANES)]
                dr = plsc.load_gather(dinv_v, [ir])
                dc = plsc.load_gather(dinv_v, [ic])
                norm_v[pl.ds(c * CH + i * LANES, LANES)] = (
                    dr * ew_v[pl.ds(c * CH + i * LANES, LANES)] * dc)
                return 0
            lax.fori_loop(0, CH // LANES, vec, 0)
            return 0
        lax.fori_loop(0, nch, chunk, 0)
        pltpu.sync_copy(norm_v, out_h.at[wid])

    return pl.kernel(
        body,
        out_type=jax.ShapeDtypeStruct((NW, nch * CH), F32),
        mesh=_mesh(),
        compiler_params=_SC_PARAMS,
        scratch_types=[
            pltpu.VMEM((nch, CH), jnp.int32),
            pltpu.VMEM((nch, CH), jnp.int32),
            pltpu.VMEM((nch * CH,), F32),
            pltpu.VMEM((n,), F32),
            pltpu.VMEM((nch * CH,), F32),
        ],
    )


@functools.lru_cache(maxsize=None)
def _make_agg_kernel(n, nch, c_feat):
    rpt = n // NS
    nf = c_feat // LANES
    nsub = 5
    assert nch % nsub == 0 and (nch // nsub) % 8 == 0
    nchs = nch // nsub

    def body(h_h, row_h, col_h, norm_h, out_h,
             row_v, col_v, norm_v, gbuf, acc, sem):
        cid = lax.axis_index("c")
        sid = lax.axis_index("s")
        wid = sid * NC + cid

        def zb(e, _):
            for f in range(nf):
                gbuf[e, pl.ds(f * LANES, LANES)] = jnp.zeros((LANES,), F32)
            return 0
        lax.fori_loop(0, CH, zb, 0)
        for k in range(rpt // CH):
            pltpu.sync_copy(gbuf, acc.at[pl.ds(sid * rpt + k * CH, CH)])
        plsc.subcore_barrier()

        for sub in range(nsub):
            pltpu.sync_copy(row_h.at[wid, pl.ds(sub * nchs, nchs)], row_v)
            pltpu.sync_copy(col_h.at[wid, pl.ds(sub * nchs, nchs)], col_v)
            pltpu.sync_copy(
                norm_h.at[wid, pl.ds(sub * nchs * CH, nchs * CH)], norm_v)
            for c in range(nchs):
                pltpu.async_copy(h_h.at[row_v.at[c]], gbuf, sem).wait()

                def scale(e, _, c=c):
                    fi = jnp.full((LANES,), c * CH + e, jnp.int32)
                    s = plsc.load_gather(norm_v, [fi])
                    for f in range(nf):
                        gbuf[e, pl.ds(f * LANES, LANES)] = (
                            gbuf[e, pl.ds(f * LANES, LANES)] * s)
                    return 0
                lax.fori_loop(0, CH, scale, 0)
                pltpu.sync_copy(gbuf, acc.at[col_v.at[c]], add=True)
        plsc.subcore_barrier()
        pltpu.sync_copy(acc.at[pl.ds(sid * rpt, rpt)],
                        out_h.at[cid, pl.ds(sid * rpt, rpt)])

    return pl.kernel(
        body,
        out_type=jax.ShapeDtypeStruct((NC, n, c_feat), F32),
        mesh=_mesh(),
        compiler_params=_SC_PARAMS,
        scratch_types=[
            pltpu.VMEM((nchs, CH), jnp.int32),
            pltpu.VMEM((nchs, CH), jnp.int32),
            pltpu.VMEM((nchs * CH,), F32),
            pltpu.VMEM((CH, c_feat), F32),
            pltpu.VMEM_SHARED((n, c_feat), F32),
            pltpu.SemaphoreType.DMA,
        ],
    )


def _tc1_body(x_ref, w_ref, degp_ref, h0_ref, h1_ref, dinv_ref, dinv2_ref):
    xb = x_ref[...]
    h0_ref[...] = jnp.dot(xb, w_ref[0], preferred_element_type=F32)
    h1_ref[...] = jnp.dot(xb, w_ref[1], preferred_element_type=F32)
    deg = degp_ref[0, :, 0] + degp_ref[1, :, 0] + 1.0
    dinv_ref[...] = lax.rsqrt(deg)[:, None]
    dinv2_ref[...] = (1.0 / deg)[:, None]


def _stats_body(n_true, bn_rows, p0_ref, p1_ref, h0_ref, h1_ref, d2_ref,
                b_ref, agg_ref, stats_ref):
    i = pl.program_id(0)
    d2 = d2_ref[...]
    hs = (h0_ref[...], h1_ref[...])
    ps = (p0_ref, p1_ref)
    ridx = i * bn_rows + lax.broadcasted_iota(jnp.int32, (bn_rows, 1), 0)
    valid = (ridx < n_true).astype(F32)
    for g in range(2):
        a = ps[g][0] + ps[g][1] + d2 * hs[g] + b_ref[g]
        agg_ref[g] = a
        am = a * valid
        s = jnp.sum(am, axis=0)
        s2 = jnp.sum(am * am, axis=0)

        @pl.when(i == 0)
        def _(s=s, s2=s2, g=g):
            stats_ref[g, 0] = s
            stats_ref[g, 1] = s2

        @pl.when(i > 0)
        def _(s=s, s2=s2, g=g):
            stats_ref[g, 0] += s
            stats_ref[g, 1] += s2


def _apply_body(n_nodes, agg_ref, stats_ref, gam_ref, bet_ref, w_ref,
                o0_ref, o1_ref):
    outs = (o0_ref, o1_ref)
    for g in range(2):
        mean = stats_ref[g, 0] / n_nodes
        var = stats_ref[g, 1] / n_nodes - mean * mean
        inv = lax.rsqrt(var + EPS) * gam_ref[g]
        a = (agg_ref[g] - mean) * inv + bet_ref[g]
        a = jnp.maximum(a, 0.0)
        outs[g][...] = jnp.dot(a, w_ref[g], preferred_element_type=F32)


def _apply_cat_body(n_nodes, agg_ref, stats_ref, gam_ref, bet_ref, w_ref,
                    o_ref):
    outs = []
    for g in range(2):
        mean = stats_ref[g, 0] / n_nodes
        var = stats_ref[g, 1] / n_nodes - mean * mean
        inv = lax.rsqrt(var + EPS) * gam_ref[g]
        a = (agg_ref[g] - mean) * inv + bet_ref[g]
        a = jnp.maximum(a, 0.0)
        outs.append(jnp.dot(a, w_ref[g], preferred_element_type=F32))
    o_ref[...] = jnp.concatenate(outs, axis=1)


def _final_body(out_c, p_ref, hcat_ref, d2_ref, b_ref, y_ref):
    d2 = d2_ref[...]
    hcat = hcat_ref[...]
    agg = p_ref[0] + p_ref[1] + d2 * hcat
    for g in range(2):
        y_ref[g] = agg[:, g * out_c:(g + 1) * out_c] + b_ref[g]


def kernel(x, edge_index, edge_attr, params):
    n, in_c = x.shape
    e = edge_index.shape[1]
    convs, bns = params["convs"], params["bns"]
    hid_c = convs[0]["W"].shape[1]
    out_c = convs[2]["W"].shape[1]

    epad = -(-e // (NW * CH * 40)) * (NW * CH * 40)
    nch = epad // (NW * CH)
    row = jnp.pad(edge_index[0], (0, epad - e)).reshape(NW, nch, CH)
    col = jnp.pad(edge_index[1], (0, epad - e)).reshape(NW, nch, CH)
    ew = jnp.pad(edge_attr, (0, epad - e)).reshape(NW, nch * CH)

    npad = -(-n // (NS * ZR)) * (NS * ZR)
    xp = jnp.pad(x, ((0, npad - n), (0, 0)))

    w0 = jnp.stack([convs[0]["W"], convs[3]["W"]])
    w1 = jnp.stack([convs[1]["W"], convs[4]["W"]])
    w2 = jnp.stack([convs[2]["W"], convs[5]["W"]])
    b0 = jnp.stack([convs[0]["b"], convs[3]["b"]])
    b1 = jnp.stack([convs[1]["b"], convs[4]["b"]])
    b2 = jnp.stack([convs[2]["b"], convs[5]["b"]])
    gam0 = jnp.stack([bns[0]["gamma"], bns[2]["gamma"]])
    bet0 = jnp.stack([bns[0]["beta"], bns[2]["beta"]])
    gam1 = jnp.stack([bns[1]["gamma"], bns[3]["gamma"]])
    bet1 = jnp.stack([bns[1]["beta"], bns[3]["beta"]])

    bn_rows = 1280
    nb = npad // bn_rows

    degp = _make_deg_kernel(npad, nch)(col, ew)

    h1_0, h1_1, dinv, dinv2 = pl.pallas_call(
        _tc1_body,
        grid=(nb,),
        in_specs=[
            pl.BlockSpec((bn_rows, in_c), lambda i: (i, 0)),
            pl.BlockSpec((2, in_c, hid_c), lambda i: (0, 0, 0)),
            pl.BlockSpec((2, bn_rows, LANES), lambda i: (0, i, 0)),
        ],
        out_specs=[
            pl.BlockSpec((bn_rows, hid_c), lambda i: (i, 0)),
            pl.BlockSpec((bn_rows, hid_c), lambda i: (i, 0)),
            pl.BlockSpec((bn_rows, 1), lambda i: (i, 0)),
            pl.BlockSpec((bn_rows, 1), lambda i: (i, 0)),
        ],
        out_shape=[
            jax.ShapeDtypeStruct((npad, hid_c), F32),
            jax.ShapeDtypeStruct((npad, hid_c), F32),
            jax.ShapeDtypeStruct((npad, 1), F32),
            jax.ShapeDtypeStruct((npad, 1), F32),
        ],
    )(xp, w0, degp)

    norm = _make_norm_kernel(npad, nch)(row, col, ew, dinv.reshape(npad))

    agg_kernel = _make_agg_kernel(npad, nch, hid_c)

    def stats_call(p0, p1, h0, h1, b):
        return pl.pallas_call(
            functools.partial(_stats_body, n, bn_rows),
            grid=(nb,),
            in_specs=[
                pl.BlockSpec((NC, bn_rows, hid_c), lambda i: (0, i, 0)),
                pl.BlockSpec((NC, bn_rows, hid_c), lambda i: (0, i, 0)),
                pl.BlockSpec((bn_rows, hid_c), lambda i: (i, 0)),
                pl.BlockSpec((bn_rows, hid_c), lambda i: (i, 0)),
                pl.BlockSpec((bn_rows, 1), lambda i: (i, 0)),
                pl.BlockSpec((2, hid_c), lambda i: (0, 0)),
            ],
            out_specs=[
                pl.BlockSpec((2, bn_rows, hid_c), lambda i: (0, i, 0)),
                pl.BlockSpec((2, 8, hid_c), lambda i: (0, 0, 0)),
            ],
            out_shape=[
                jax.ShapeDtypeStruct((2, npad, hid_c), F32),
                jax.ShapeDtypeStruct((2, 8, hid_c), F32),
            ],
        )(p0, p1, h0, h1, dinv2, b)

    def apply_call(agg, stats, gam, bet, w, co):
        return pl.pallas_call(
            functools.partial(_apply_body, float(n)),
            grid=(nb,),
            in_specs=[
                pl.BlockSpec((2, bn_rows, hid_c), lambda i: (0, i, 0)),
                pl.BlockSpec((2, 8, hid_c), lambda i: (0, 0, 0)),
                pl.BlockSpec((2, hid_c), lambda i: (0, 0)),
                pl.BlockSpec((2, hid_c), lambda i: (0, 0)),
                pl.BlockSpec((2, hid_c, co), lambda i: (0, 0, 0)),
            ],
            out_specs=[
                pl.BlockSpec((bn_rows, co), lambda i: (i, 0)),
                pl.BlockSpec((bn_rows, co), lambda i: (i, 0)),
            ],
            out_shape=[
                jax.ShapeDtypeStruct((npad, co), F32),
                jax.ShapeDtypeStruct((npad, co), F32),
            ],
        )(agg, stats, gam, bet, w)

    p1_0 = agg_kernel(h1_0, row, col, norm)
    p1_1 = agg_kernel(h1_1, row, col, norm)
    agg1, stats1 = stats_call(p1_0, p1_1, h1_0, h1_1, b0)
    h2_0, h2_1 = apply_call(agg1, stats1, gam0, bet0, w1, hid_c)

    p2_0 = agg_kernel(h2_0, row, col, norm)
    p2_1 = agg_kernel(h2_1, row, col, norm)
    agg2, stats2 = stats_call(p2_0, p2_1, h2_0, h2_1, b1)
    h3cat = pl.pallas_call(
        functools.partial(_apply_cat_body, float(n)),
        grid=(nb,),
        in_specs=[
            pl.BlockSpec((2, bn_rows, hid_c), lambda i: (0, i, 0)),
            pl.BlockSpec((2, 8, hid_c), lambda i: (0, 0, 0)),
            pl.BlockSpec((2, hid_c), lambda i: (0, 0)),
            pl.BlockSpec((2, hid_c), lambda i: (0, 0)),
            pl.BlockSpec((2, hid_c, out_c), lambda i: (0, 0, 0)),
        ],
        out_specs=pl.BlockSpec((bn_rows, 2 * out_c), lambda i: (i, 0)),
        out_shape=jax.ShapeDtypeStruct((npad, 2 * out_c), F32),
    )(agg2, stats2, gam1, bet1, w2)

    p3 = _make_agg_kernel(npad, nch, 2 * out_c)(h3cat, row, col, norm)

    y = pl.pallas_call(
        functools.partial(_final_body, out_c),
        grid=(nb,),
        in_specs=[
            pl.BlockSpec((NC, bn_rows, 2 * out_c), lambda i: (0, i, 0)),
            pl.BlockSpec((bn_rows, 2 * out_c), lambda i: (i, 0)),
            pl.BlockSpec((bn_rows, 1), lambda i: (i, 0)),
            pl.BlockSpec((2, out_c), lambda i: (0, 0)),
        ],
        out_specs=pl.BlockSpec((2, bn_rows, out_c), lambda i: (0, i, 0)),
        out_shape=jax.ShapeDtypeStruct((2, npad, out_c), F32),
    )(p3, h3cat, dinv2, b2)

    return y[:, :n]


# speedup vs baseline: 5.8164x; 1.2195x over previous
import functools

import jax
import jax.numpy as jnp
from jax import lax
from jax.experimental import pallas as pl
from jax.experimental.pallas import tpu as pltpu
from jax.experimental.pallas import tpu_sc as plsc

F32 = jnp.float32
EPS = 1e-5
NC, NS, LANES = 2, 16, 16
NW = NC * NS
CH = 128
ZR = 160


def _mesh():
    return plsc.VectorSubcoreMesh(core_axis_name="c", subcore_axis_name="s")


_SC_PARAMS = pltpu.CompilerParams(needs_layout_passes=False)


@functools.lru_cache(maxsize=None)
def _make_deg_kernel(n, nch):
    rpt = n // NS

    def body(col_h, ew_h, out_h, col_v, ew_v, vbuf, acc, sem):
        cid = lax.axis_index("c")
        sid = lax.axis_index("s")
        wid = sid * NC + cid
        pltpu.sync_copy(col_h.at[wid], col_v)
        pltpu.sync_copy(ew_h.at[wid], ew_v)

        def zb(e, _):
            vbuf[e, :] = jnp.zeros((LANES,), F32)
            return 0
        lax.fori_loop(0, CH, zb, 0)
        for k in range(rpt // CH):
            pltpu.sync_copy(vbuf, acc.at[pl.ds(sid * rpt + k * CH, CH)])
        plsc.subcore_barrier()

        for c in range(nch):
            def fill(e, _, c=c):
                fi = jnp.full((LANES,), c * CH + e, jnp.int32)
                vbuf[e, :] = plsc.load_gather(ew_v, [fi])
                return 0
            lax.fori_loop(0, CH, fill, 0)
            pltpu.sync_copy(vbuf, acc.at[col_v.at[c]], add=True)
        plsc.subcore_barrier()
        pltpu.sync_copy(acc.at[pl.ds(sid * rpt, rpt)],
                        out_h.at[cid, pl.ds(sid * rpt, rpt)])

    return pl.kernel(
        body,
        out_type=jax.ShapeDtypeStruct((NC, n, LANES), F32),
        mesh=_mesh(),
        compiler_params=_SC_PARAMS,
        scratch_types=[
            pltpu.VMEM((nch, CH), jnp.int32),
            pltpu.VMEM((nch * CH,), F32),
            pltpu.VMEM((CH, LANES), F32),
            pltpu.VMEM_SHARED((n, LANES), F32),
            pltpu.SemaphoreType.DMA,
        ],
    )


@functools.lru_cache(maxsize=None)
def _make_norm_kernel(n, nch):
    def body(row_h, col_h, ew_h, dinv_h, out_h, row_v, col_v, ew_v, dinv_v,
             norm_v):
        cid = lax.axis_index("c")
        sid = lax.axis_index("s")
        wid = sid * NC + cid
        pltpu.sync_copy(row_h.at[wid], row_v)
        pltpu.sync_copy(col_h.at[wid], col_v)
        pltpu.sync_copy(ew_h.at[wid], ew_v)
        pltpu.sync_copy(dinv_h, dinv_v)

        def chunk(c, _):
            def vec(i, _):
                ir = row_v[c, pl.ds(i * LANES, LANES)]
                ic = col_v[c, pl.ds(i * LANES, LANES)]
                dr = plsc.load_gather(dinv_v, [ir])
                dc = plsc.load_gather(dinv_v, [ic])
                norm_v[pl.ds(c * CH + i * LANES, LANES)] = (
                    dr * ew_v[pl.ds(c * CH + i * LANES, LANES)] * dc)
                return 0
            lax.fori_loop(0, CH // LANES, vec, 0)
            return 0
        lax.fori_loop(0, nch, chunk, 0)
        pltpu.sync_copy(norm_v, out_h.at[wid])

    return pl.kernel(
        body,
        out_type=jax.ShapeDtypeStruct((NW, nch * CH), F32),
        mesh=_mesh(),
        compiler_params=_SC_PARAMS,
        scratch_types=[
            pltpu.VMEM((nch, CH), jnp.int32),
            pltpu.VMEM((nch, CH), jnp.int32),
            pltpu.VMEM((nch * CH,), F32),
            pltpu.VMEM((n,), F32),
            pltpu.VMEM((nch * CH,), F32),
        ],
    )


@functools.lru_cache(maxsize=None)
def _make_agg_kernel(n, nch, c_feat):
    rpt = n // NS
    nf = c_feat // LANES
    nsub = 5
    assert nch % nsub == 0 and (nch // nsub) % 8 == 0
    nchs = nch // nsub

    def body(h_h, row_h, col_h, norm_h, out_h,
             row_v, col_v, norm_v, gbuf0, gbuf1, acc, sem0, sem1, sem2):
        cid = lax.axis_index("c")
        sid = lax.axis_index("s")
        wid = sid * NC + cid
        gbufs = (gbuf0, gbuf1)
        sems = (sem0, sem1)

        def zb(e, _):
            for f in range(nf):
                gbuf0[e, pl.ds(f * LANES, LANES)] = jnp.zeros((LANES,), F32)
            return 0
        lax.fori_loop(0, CH, zb, 0)
        for k in range(rpt // CH):
            pltpu.sync_copy(gbuf0, acc.at[pl.ds(sid * rpt + k * CH, CH)])
        plsc.subcore_barrier()

        for sub in range(nsub):
            pltpu.sync_copy(row_h.at[wid, pl.ds(sub * nchs, nchs)], row_v)
            pltpu.sync_copy(col_h.at[wid, pl.ds(sub * nchs, nchs)], col_v)
            pltpu.sync_copy(
                norm_h.at[wid, pl.ds(sub * nchs * CH, nchs * CH)], norm_v)
            pending = pltpu.async_copy(h_h.at[row_v.at[0]], gbuf0, sem0)
            for c in range(nchs):
                pending.wait()
                if c + 1 < nchs:
                    pending = pltpu.async_copy(
                        h_h.at[row_v.at[c + 1]], gbufs[(c + 1) % 2],
                        sems[(c + 1) % 2])
                buf = gbufs[c % 2]

                def scale(e, _, c=c, buf=buf):
                    fi = jnp.full((LANES,), c * CH + e, jnp.int32)
                    s = plsc.load_gather(norm_v, [fi])
                    for f in range(nf):
                        buf[e, pl.ds(f * LANES, LANES)] = (
                            buf[e, pl.ds(f * LANES, LANES)] * s)
                    return 0
                lax.fori_loop(0, CH, scale, 0)
                pltpu.async_copy(
                    buf, acc.at[col_v.at[c]], sem2, add=True).wait()
        plsc.subcore_barrier()
        pltpu.sync_copy(acc.at[pl.ds(sid * rpt, rpt)],
                        out_h.at[cid, pl.ds(sid * rpt, rpt)])

    return pl.kernel(
        body,
        out_type=jax.ShapeDtypeStruct((NC, n, c_feat), F32),
        mesh=_mesh(),
        compiler_params=_SC_PARAMS,
        scratch_types=[
            pltpu.VMEM((nchs, CH), jnp.int32),
            pltpu.VMEM((nchs, CH), jnp.int32),
            pltpu.VMEM((nchs * CH,), F32),
            pltpu.VMEM((CH, c_feat), F32),
            pltpu.VMEM((CH, c_feat), F32),
            pltpu.VMEM_SHARED((n, c_feat), F32),
            pltpu.SemaphoreType.DMA,
            pltpu.SemaphoreType.DMA,
            pltpu.SemaphoreType.DMA,
        ],
    )


def _tc1_body(x_ref, w_ref, degp_ref, h0_ref, h1_ref, dinv_ref, dinv2_ref):
    xb = x_ref[...]
    h0_ref[...] = jnp.dot(xb, w_ref[0], preferred_element_type=F32)
    h1_ref[...] = jnp.dot(xb, w_ref[1], preferred_element_type=F32)
    deg = degp_ref[0, :, 0] + degp_ref[1, :, 0] + 1.0
    dinv_ref[...] = lax.rsqrt(deg)[:, None]
    dinv2_ref[...] = (1.0 / deg)[:, None]


def _stats_body(n_true, bn_rows, p0_ref, p1_ref, h0_ref, h1_ref, d2_ref,
                b_ref, agg_ref, stats_ref):
    i = pl.program_id(0)
    d2 = d2_ref[...]
    hs = (h0_ref[...], h1_ref[...])
    ps = (p0_ref, p1_ref)
    ridx = i * bn_rows + lax.broadcasted_iota(jnp.int32, (bn_rows, 1), 0)
    valid = (ridx < n_true).astype(F32)
    for g in range(2):
        a = ps[g][0] + ps[g][1] + d2 * hs[g] + b_ref[g]
        agg_ref[g] = a
        am = a * valid
        s = jnp.sum(am, axis=0)
        s2 = jnp.sum(am * am, axis=0)

        @pl.when(i == 0)
        def _(s=s, s2=s2, g=g):
            stats_ref[g, 0] = s
            stats_ref[g, 1] = s2

        @pl.when(i > 0)
        def _(s=s, s2=s2, g=g):
            stats_ref[g, 0] += s
            stats_ref[g, 1] += s2


def _apply_body(n_nodes, agg_ref, stats_ref, gam_ref, bet_ref, w_ref,
                o0_ref, o1_ref):
    outs = (o0_ref, o1_ref)
    for g in range(2):
        mean = stats_ref[g, 0] / n_nodes
        var = stats_ref[g, 1] / n_nodes - mean * mean
        inv = lax.rsqrt(var + EPS) * gam_ref[g]
        a = (agg_ref[g] - mean) * inv + bet_ref[g]
        a = jnp.maximum(a, 0.0)
        outs[g][...] = jnp.dot(a, w_ref[g], preferred_element_type=F32)


def _apply_cat_body(n_nodes, agg_ref, stats_ref, gam_ref, bet_ref, w_ref,
                    o_ref):
    outs = []
    for g in range(2):
        mean = stats_ref[g, 0] / n_nodes
        var = stats_ref[g, 1] / n_nodes - mean * mean
        inv = lax.rsqrt(var + EPS) * gam_ref[g]
        a = (agg_ref[g] - mean) * inv + bet_ref[g]
        a = jnp.maximum(a, 0.0)
        outs.append(jnp.dot(a, w_ref[g], preferred_element_type=F32))
    o_ref[...] = jnp.concatenate(outs, axis=1)


def _final_body(out_c, p_ref, hcat_ref, d2_ref, b_ref, y_ref):
    d2 = d2_ref[...]
    hcat = hcat_ref[...]
    agg = p_ref[0] + p_ref[1] + d2 * hcat
    for g in range(2):
        y_ref[g] = agg[:, g * out_c:(g + 1) * out_c] + b_ref[g]


def kernel(x, edge_index, edge_attr, params):
    n, in_c = x.shape
    e = edge_index.shape[1]
    convs, bns = params["convs"], params["bns"]
    hid_c = convs[0]["W"].shape[1]
    out_c = convs[2]["W"].shape[1]

    epad = -(-e // (NW * CH * 40)) * (NW * CH * 40)
    nch = epad // (NW * CH)
    row = jnp.pad(edge_index[0], (0, epad - e)).reshape(NW, nch, CH)
    col = jnp.pad(edge_index[1], (0, epad - e)).reshape(NW, nch, CH)
    ew = jnp.pad(edge_attr, (0, epad - e)).reshape(NW, nch * CH)

    npad = -(-n // (NS * ZR)) * (NS * ZR)
    xp = jnp.pad(x, ((0, npad - n), (0, 0)))

    w0 = jnp.stack([convs[0]["W"], convs[3]["W"]])
    w1 = jnp.stack([convs[1]["W"], convs[4]["W"]])
    w2 = jnp.stack([convs[2]["W"], convs[5]["W"]])
    b0 = jnp.stack([convs[0]["b"], convs[3]["b"]])
    b1 = jnp.stack([convs[1]["b"], convs[4]["b"]])
    b2 = jnp.stack([convs[2]["b"], convs[5]["b"]])
    gam0 = jnp.stack([bns[0]["gamma"], bns[2]["gamma"]])
    bet0 = jnp.stack([bns[0]["beta"], bns[2]["beta"]])
    gam1 = jnp.stack([bns[1]["gamma"], bns[3]["gamma"]])
    bet1 = jnp.stack([bns[1]["beta"], bns[3]["beta"]])

    bn_rows = 1280
    nb = npad // bn_rows

    degp = _make_deg_kernel(npad, nch)(col, ew)

    h1_0, h1_1, dinv, dinv2 = pl.pallas_call(
        _tc1_body,
        grid=(nb,),
        in_specs=[
            pl.BlockSpec((bn_rows, in_c), lambda i: (i, 0)),
            pl.BlockSpec((2, in_c, hid_c), lambda i: (0, 0, 0)),
            pl.BlockSpec((2, bn_rows, LANES), lambda i: (0, i, 0)),
        ],
        out_specs=[
            pl.BlockSpec((bn_rows, hid_c), lambda i: (i, 0)),
            pl.BlockSpec((bn_rows, hid_c), lambda i: (i, 0)),
            pl.BlockSpec((bn_rows, 1), lambda i: (i, 0)),
            pl.BlockSpec((bn_rows, 1), lambda i: (i, 0)),
        ],
        out_shape=[
            jax.ShapeDtypeStruct((npad, hid_c), F32),
            jax.ShapeDtypeStruct((npad, hid_c), F32),
            jax.ShapeDtypeStruct((npad, 1), F32),
            jax.ShapeDtypeStruct((npad, 1), F32),
        ],
    )(xp, w0, degp)

    norm = _make_norm_kernel(npad, nch)(row, col, ew, dinv.reshape(npad))

    agg_kernel = _make_agg_kernel(npad, nch, hid_c)

    def stats_call(p0, p1, h0, h1, b):
        return pl.pallas_call(
            functools.partial(_stats_body, n, bn_rows),
            grid=(nb,),
            in_specs=[
                pl.BlockSpec((NC, bn_rows, hid_c), lambda i: (0, i, 0)),
                pl.BlockSpec((NC, bn_rows, hid_c), lambda i: (0, i, 0)),
                pl.BlockSpec((bn_rows, hid_c), lambda i: (i, 0)),
                pl.BlockSpec((bn_rows, hid_c), lambda i: (i, 0)),
                pl.BlockSpec((bn_rows, 1), lambda i: (i, 0)),
                pl.BlockSpec((2, hid_c), lambda i: (0, 0)),
            ],
            out_specs=[
                pl.BlockSpec((2, bn_rows, hid_c), lambda i: (0, i, 0)),
                pl.BlockSpec((2, 8, hid_c), lambda i: (0, 0, 0)),
            ],
            out_shape=[
                jax.ShapeDtypeStruct((2, npad, hid_c), F32),
                jax.ShapeDtypeStruct((2, 8, hid_c), F32),
            ],
        )(p0, p1, h0, h1, dinv2, b)

    def apply_call(agg, stats, gam, bet, w, co):
        return pl.pallas_call(
            functools.partial(_apply_body, float(n)),
            grid=(nb,),
            in_specs=[
                pl.BlockSpec((2, bn_rows, hid_c), lambda i: (0, i, 0)),
                pl.BlockSpec((2, 8, hid_c), lambda i: (0, 0, 0)),
                pl.BlockSpec((2, hid_c), lambda i: (0, 0)),
                pl.BlockSpec((2, hid_c), lambda i: (0, 0)),
                pl.BlockSpec((2, hid_c, co), lambda i: (0, 0, 0)),
            ],
            out_specs=[
                pl.BlockSpec((bn_rows, co), lambda i: (i, 0)),
                pl.BlockSpec((bn_rows, co), lambda i: (i, 0)),
            ],
            out_shape=[
                jax.ShapeDtypeStruct((npad, co), F32),
                jax.ShapeDtypeStruct((npad, co), F32),
            ],
        )(agg, stats, gam, bet, w)

    p1_0 = agg_kernel(h1_0, row, col, norm)
    p1_1 = agg_kernel(h1_1, row, col, norm)
    agg1, stats1 = stats_call(p1_0, p1_1, h1_0, h1_1, b0)
    h2_0, h2_1 = apply_call(agg1, stats1, gam0, bet0, w1, hid_c)

    p2_0 = agg_kernel(h2_0, row, col, norm)
    p2_1 = agg_kernel(h2_1, row, col, norm)
    agg2, stats2 = stats_call(p2_0, p2_1, h2_0, h2_1, b1)
    h3cat = pl.pallas_call(
        functools.partial(_apply_cat_body, float(n)),
        grid=(nb,),
        in_specs=[
            pl.BlockSpec((2, bn_rows, hid_c), lambda i: (0, i, 0)),
            pl.BlockSpec((2, 8, hid_c), lambda i: (0, 0, 0)),
            pl.BlockSpec((2, hid_c), lambda i: (0, 0)),
            pl.BlockSpec((2, hid_c), lambda i: (0, 0)),
            pl.BlockSpec((2, hid_c, out_c), lambda i: (0, 0, 0)),
        ],
        out_specs=pl.BlockSpec((bn_rows, 2 * out_c), lambda i: (i, 0)),
        out_shape=jax.ShapeDtypeStruct((npad, 2 * out_c), F32),
    )(agg2, stats2, gam1, bet1, w2)

    p3 = _make_agg_kernel(npad, nch, 2 * out_c)(h3cat, row, col, norm)

    y = pl.pallas_call(
        functools.partial(_final_body, out_c),
        grid=(nb,),
        in_specs=[
            pl.BlockSpec((NC, bn_rows, 2 * out_c), lambda i: (0, i, 0)),
            pl.BlockSpec((bn_rows, 2 * out_c), lambda i: (i, 0)),
            pl.BlockSpec((bn_rows, 1), lambda i: (i, 0)),
            pl.BlockSpec((2, out_c), lambda i: (0, 0)),
        ],
        out_specs=pl.BlockSpec((2, bn_rows, out_c), lambda i: (0, i, 0)),
        out_shape=jax.ShapeDtypeStruct((2, npad, out_c), F32),
    )(p3, h3cat, dinv2, b2)

    return y[:, :n]


# speedup vs baseline: 5.8282x; 1.0020x over previous
import functools

import jax
import jax.numpy as jnp
from jax import lax
from jax.experimental import pallas as pl
from jax.experimental.pallas import tpu as pltpu
from jax.experimental.pallas import tpu_sc as plsc

F32 = jnp.float32
EPS = 1e-5
NC, NS, LANES = 2, 16, 16
NW = NC * NS
CH = 128
ZR = 160


def _mesh():
    return plsc.VectorSubcoreMesh(core_axis_name="c", subcore_axis_name="s")


_SC_PARAMS = pltpu.CompilerParams(needs_layout_passes=False)


@functools.lru_cache(maxsize=None)
def _make_deg_kernel(n, nch):
    rpt = n // NS

    def body(col_h, ew_h, out_h, col_v, ew_v, vbuf, acc, sem):
        cid = lax.axis_index("c")
        sid = lax.axis_index("s")
        wid = sid * NC + cid
        pltpu.sync_copy(col_h.at[wid], col_v)
        pltpu.sync_copy(ew_h.at[wid], ew_v)

        def zb(e, _):
            vbuf[e, :] = jnp.zeros((LANES,), F32)
            return 0
        lax.fori_loop(0, CH, zb, 0)
        for k in range(rpt // CH):
            pltpu.sync_copy(vbuf, acc.at[pl.ds(sid * rpt + k * CH, CH)])
        plsc.subcore_barrier()

        for c in range(nch):
            def fill(e, _, c=c):
                fi = jnp.full((LANES,), c * CH + e, jnp.int32)
                vbuf[e, :] = plsc.load_gather(ew_v, [fi])
                return 0
            lax.fori_loop(0, CH, fill, 0)
            pltpu.sync_copy(vbuf, acc.at[col_v.at[c]], add=True)
        plsc.subcore_barrier()
        pltpu.sync_copy(acc.at[pl.ds(sid * rpt, rpt)],
                        out_h.at[cid, pl.ds(sid * rpt, rpt)])

    return pl.kernel(
        body,
        out_type=jax.ShapeDtypeStruct((NC, n, LANES), F32),
        mesh=_mesh(),
        compiler_params=_SC_PARAMS,
        scratch_types=[
            pltpu.VMEM((nch, CH), jnp.int32),
            pltpu.VMEM((nch * CH,), F32),
            pltpu.VMEM((CH, LANES), F32),
            pltpu.VMEM_SHARED((n, LANES), F32),
            pltpu.SemaphoreType.DMA,
        ],
    )


@functools.lru_cache(maxsize=None)
def _make_norm_kernel(n, nch):
    def body(row_h, col_h, ew_h, dinv_h, out_h, row_v, col_v, ew_v, dinv_v,
             norm_v):
        cid = lax.axis_index("c")
        sid = lax.axis_index("s")
        wid = sid * NC + cid
        pltpu.sync_copy(row_h.at[wid], row_v)
        pltpu.sync_copy(col_h.at[wid], col_v)
        pltpu.sync_copy(ew_h.at[wid], ew_v)
        pltpu.sync_copy(dinv_h, dinv_v)

        def chunk(c, _):
            def vec(i, _):
                ir = row_v[c, pl.ds(i * LANES, LANES)]
                ic = col_v[c, pl.ds(i * LANES, LANES)]
                dr = plsc.load_gather(dinv_v, [ir])
                dc = plsc.load_gather(dinv_v, [ic])
                norm_v[pl.ds(c * CH + i * LANES, LANES)] = (
                    dr * ew_v[pl.ds(c * CH + i * LANES, LANES)] * dc)
                return 0
            lax.fori_loop(0, CH // LANES, vec, 0)
            return 0
        lax.fori_loop(0, nch, chunk, 0)
        pltpu.sync_copy(norm_v, out_h.at[wid])

    return pl.kernel(
        body,
        out_type=jax.ShapeDtypeStruct((NW, nch * CH), F32),
        mesh=_mesh(),
        compiler_params=_SC_PARAMS,
        scratch_types=[
            pltpu.VMEM((nch, CH), jnp.int32),
            pltpu.VMEM((nch, CH), jnp.int32),
            pltpu.VMEM((nch * CH,), F32),
            pltpu.VMEM((n,), F32),
            pltpu.VMEM((nch * CH,), F32),
        ],
    )


@functools.lru_cache(maxsize=None)
def _make_agg_kernel(n, nch, c_feat):
    rpt = n // NS
    nf = c_feat // LANES
    nsub = 5
    assert nch % nsub == 0 and (nch // nsub) % 8 == 0
    nchs = nch // nsub

    def body(h_h, row_h, col_h, norm_h, out_h,
             row_v, col_v, norm_v, gbuf0, gbuf1, acc, sem0, sem1, sem2,
             sem3):
        cid = lax.axis_index("c")
        sid = lax.axis_index("s")
        wid = sid * NC + cid
        gbufs = (gbuf0, gbuf1)
        sems = (sem0, sem1)

        def zb(e, _):
            for f in range(nf):
                gbuf0[e, pl.ds(f * LANES, LANES)] = jnp.zeros((LANES,), F32)
            return 0
        lax.fori_loop(0, CH, zb, 0)
        for k in range(rpt // CH):
            pltpu.sync_copy(gbuf0, acc.at[pl.ds(sid * rpt + k * CH, CH)])
        plsc.subcore_barrier()

        for sub in range(nsub):
            pltpu.sync_copy(row_h.at[wid, pl.ds(sub * nchs, nchs)], row_v)
            pltpu.sync_copy(col_h.at[wid, pl.ds(sub * nchs, nchs)], col_v)
            pltpu.sync_copy(
                norm_h.at[wid, pl.ds(sub * nchs * CH, nchs * CH)], norm_v)
            ssems = (sem2, sem3)
            pend_sc = [None, None]
            pending = pltpu.async_copy(h_h.at[row_v.at[0]], gbuf0, sem0)
            for c in range(nchs):
                pending.wait()
                if c + 1 < nchs:
                    if pend_sc[(c + 1) % 2] is not None:
                        pend_sc[(c + 1) % 2].wait()
                        pend_sc[(c + 1) % 2] = None
                    pending = pltpu.async_copy(
                        h_h.at[row_v.at[c + 1]], gbufs[(c + 1) % 2],
                        sems[(c + 1) % 2])
                buf = gbufs[c % 2]

                def scale(e, _, c=c, buf=buf):
                    fi = jnp.full((LANES,), c * CH + e, jnp.int32)
                    s = plsc.load_gather(norm_v, [fi])
                    for f in range(nf):
                        buf[e, pl.ds(f * LANES, LANES)] = (
                            buf[e, pl.ds(f * LANES, LANES)] * s)
                    return 0
                lax.fori_loop(0, CH, scale, 0)
                pend_sc[c % 2] = pltpu.async_copy(
                    buf, acc.at[col_v.at[c]], ssems[c % 2], add=True)
            for b in range(2):
                if pend_sc[b] is not None:
                    pend_sc[b].wait()
        plsc.subcore_barrier()
        pltpu.sync_copy(acc.at[pl.ds(sid * rpt, rpt)],
                        out_h.at[cid, pl.ds(sid * rpt, rpt)])

    return pl.kernel(
        body,
        out_type=jax.ShapeDtypeStruct((NC, n, c_feat), F32),
        mesh=_mesh(),
        compiler_params=_SC_PARAMS,
        scratch_types=[
            pltpu.VMEM((nchs, CH), jnp.int32),
            pltpu.VMEM((nchs, CH), jnp.int32),
            pltpu.VMEM((nchs * CH,), F32),
            pltpu.VMEM((CH, c_feat), F32),
            pltpu.VMEM((CH, c_feat), F32),
            pltpu.VMEM_SHARED((n, c_feat), F32),
            pltpu.SemaphoreType.DMA,
            pltpu.SemaphoreType.DMA,
            pltpu.SemaphoreType.DMA,
            pltpu.SemaphoreType.DMA,
        ],
    )


def _tc1_body(x_ref, w_ref, degp_ref, h0_ref, h1_ref, dinv_ref, dinv2_ref):
    xb = x_ref[...]
    h0_ref[...] = jnp.dot(xb, w_ref[0], preferred_element_type=F32)
    h1_ref[...] = jnp.dot(xb, w_ref[1], preferred_element_type=F32)
    deg = degp_ref[0, :, 0] + degp_ref[1, :, 0] + 1.0
    dinv_ref[...] = lax.rsqrt(deg)[:, None]
    dinv2_ref[...] = (1.0 / deg)[:, None]


def _stats_body(n_true, bn_rows, p0_ref, p1_ref, h0_ref, h1_ref, d2_ref,
                b_ref, agg_ref, stats_ref):
    i = pl.program_id(0)
    d2 = d2_ref[...]
    hs = (h0_ref[...], h1_ref[...])
    ps = (p0_ref, p1_ref)
    ridx = i * bn_rows + lax.broadcasted_iota(jnp.int32, (bn_rows, 1), 0)
    valid = (ridx < n_true).astype(F32)
    for g in range(2):
        a = ps[g][0] + ps[g][1] + d2 * hs[g] + b_ref[g]
        agg_ref[g] = a
        am = a * valid
        s = jnp.sum(am, axis=0)
        s2 = jnp.sum(am * am, axis=0)

        @pl.when(i == 0)
        def _(s=s, s2=s2, g=g):
            stats_ref[g, 0] = s
            stats_ref[g, 1] = s2

        @pl.when(i > 0)
        def _(s=s, s2=s2, g=g):
            stats_ref[g, 0] += s
            stats_ref[g, 1] += s2


def _apply_body(n_nodes, agg_ref, stats_ref, gam_ref, bet_ref, w_ref,
                o0_ref, o1_ref):
    outs = (o0_ref, o1_ref)
    for g in range(2):
        mean = stats_ref[g, 0] / n_nodes
        var = stats_ref[g, 1] / n_nodes - mean * mean
        inv = lax.rsqrt(var + EPS) * gam_ref[g]
        a = (agg_ref[g] - mean) * inv + bet_ref[g]
        a = jnp.maximum(a, 0.0)
        outs[g][...] = jnp.dot(a, w_ref[g], preferred_element_type=F32)


def _apply_cat_body(n_nodes, agg_ref, stats_ref, gam_ref, bet_ref, w_ref,
                    o_ref):
    outs = []
    for g in range(2):
        mean = stats_ref[g, 0] / n_nodes
        var = stats_ref[g, 1] / n_nodes - mean * mean
        inv = lax.rsqrt(var + EPS) * gam_ref[g]
        a = (agg_ref[g] - mean) * inv + bet_ref[g]
        a = jnp.maximum(a, 0.0)
        outs.append(jnp.dot(a, w_ref[g], preferred_element_type=F32))
    o_ref[...] = jnp.concatenate(outs, axis=1)


def _final_body(out_c, p_ref, hcat_ref, d2_ref, b_ref, y_ref):
    d2 = d2_ref[...]
    hcat = hcat_ref[...]
    agg = p_ref[0] + p_ref[1] + d2 * hcat
    for g in range(2):
        y_ref[g] = agg[:, g * out_c:(g + 1) * out_c] + b_ref[g]


def kernel(x, edge_index, edge_attr, params):
    n, in_c = x.shape
    e = edge_index.shape[1]
    convs, bns = params["convs"], params["bns"]
    hid_c = convs[0]["W"].shape[1]
    out_c = convs[2]["W"].shape[1]

    epad = -(-e // (NW * CH * 40)) * (NW * CH * 40)
    nch = epad // (NW * CH)
    row = jnp.pad(edge_index[0], (0, epad - e)).reshape(NW, nch, CH)
    col = jnp.pad(edge_index[1], (0, epad - e)).reshape(NW, nch, CH)
    ew = jnp.pad(edge_attr, (0, epad - e)).reshape(NW, nch * CH)

    npad = -(-n // (NS * ZR)) * (NS * ZR)
    xp = jnp.pad(x, ((0, npad - n), (0, 0)))

    w0 = jnp.stack([convs[0]["W"], convs[3]["W"]])
    w1 = jnp.stack([convs[1]["W"], convs[4]["W"]])
    w2 = jnp.stack([convs[2]["W"], convs[5]["W"]])
    b0 = jnp.stack([convs[0]["b"], convs[3]["b"]])
    b1 = jnp.stack([convs[1]["b"], convs[4]["b"]])
    b2 = jnp.stack([convs[2]["b"], convs[5]["b"]])
    gam0 = jnp.stack([bns[0]["gamma"], bns[2]["gamma"]])
    bet0 = jnp.stack([bns[0]["beta"], bns[2]["beta"]])
    gam1 = jnp.stack([bns[1]["gamma"], bns[3]["gamma"]])
    bet1 = jnp.stack([bns[1]["beta"], bns[3]["beta"]])

    bn_rows = 1280
    nb = npad // bn_rows

    degp = _make_deg_kernel(npad, nch)(col, ew)

    h1_0, h1_1, dinv, dinv2 = pl.pallas_call(
        _tc1_body,
        grid=(nb,),
        in_specs=[
            pl.BlockSpec((bn_rows, in_c), lambda i: (i, 0)),
            pl.BlockSpec((2, in_c, hid_c), lambda i: (0, 0, 0)),
            pl.BlockSpec((2, bn_rows, LANES), lambda i: (0, i, 0)),
        ],
        out_specs=[
            pl.BlockSpec((bn_rows, hid_c), lambda i: (i, 0)),
            pl.BlockSpec((bn_rows, hid_c), lambda i: (i, 0)),
            pl.BlockSpec((bn_rows, 1), lambda i: (i, 0)),
            pl.BlockSpec((bn_rows, 1), lambda i: (i, 0)),
        ],
        out_shape=[
            jax.ShapeDtypeStruct((npad, hid_c), F32),
            jax.ShapeDtypeStruct((npad, hid_c), F32),
            jax.ShapeDtypeStruct((npad, 1), F32),
            jax.ShapeDtypeStruct((npad, 1), F32),
        ],
    )(xp, w0, degp)

    norm = _make_norm_kernel(npad, nch)(row, col, ew, dinv.reshape(npad))

    agg_kernel = _make_agg_kernel(npad, nch, hid_c)

    def stats_call(p0, p1, h0, h1, b):
        return pl.pallas_call(
            functools.partial(_stats_body, n, bn_rows),
            grid=(nb,),
            in_specs=[
                pl.BlockSpec((NC, bn_rows, hid_c), lambda i: (0, i, 0)),
                pl.BlockSpec((NC, bn_rows, hid_c), lambda i: (0, i, 0)),
                pl.BlockSpec((bn_rows, hid_c), lambda i: (i, 0)),
                pl.BlockSpec((bn_rows, hid_c), lambda i: (i, 0)),
                pl.BlockSpec((bn_rows, 1), lambda i: (i, 0)),
                pl.BlockSpec((2, hid_c), lambda i: (0, 0)),
            ],
            out_specs=[
                pl.BlockSpec((2, bn_rows, hid_c), lambda i: (0, i, 0)),
                pl.BlockSpec((2, 8, hid_c), lambda i: (0, 0, 0)),
            ],
            out_shape=[
                jax.ShapeDtypeStruct((2, npad, hid_c), F32),
                jax.ShapeDtypeStruct((2, 8, hid_c), F32),
            ],
        )(p0, p1, h0, h1, dinv2, b)

    def apply_call(agg, stats, gam, bet, w, co):
        return pl.pallas_call(
            functools.partial(_apply_body, float(n)),
            grid=(nb,),
            in_specs=[
                pl.BlockSpec((2, bn_rows, hid_c), lambda i: (0, i, 0)),
                pl.BlockSpec((2, 8, hid_c), lambda i: (0, 0, 0)),
                pl.BlockSpec((2, hid_c), lambda i: (0, 0)),
                pl.BlockSpec((2, hid_c), lambda i: (0, 0)),
                pl.BlockSpec((2, hid_c, co), lambda i: (0, 0, 0)),
            ],
            out_specs=[
                pl.BlockSpec((bn_rows, co), lambda i: (i, 0)),
                pl.BlockSpec((bn_rows, co), lambda i: (i, 0)),
            ],
            out_shape=[
                jax.ShapeDtypeStruct((npad, co), F32),
                jax.ShapeDtypeStruct((npad, co), F32),
            ],
        )(agg, stats, gam, bet, w)

    p1_0 = agg_kernel(h1_0, row, col, norm)
    p1_1 = agg_kernel(h1_1, row, col, norm)
    agg1, stats1 = stats_call(p1_0, p1_1, h1_0, h1_1, b0)
    h2_0, h2_1 = apply_call(agg1, stats1, gam0, bet0, w1, hid_c)

    p2_0 = agg_kernel(h2_0, row, col, norm)
    p2_1 = agg_kernel(h2_1, row, col, norm)
    agg2, stats2 = stats_call(p2_0, p2_1, h2_0, h2_1, b1)
    h3cat = pl.pallas_call(
        functools.partial(_apply_cat_body, float(n)),
        grid=(nb,),
        in_specs=[
            pl.BlockSpec((2, bn_rows, hid_c), lambda i: (0, i, 0)),
            pl.BlockSpec((2, 8, hid_c), lambda i: (0, 0, 0)),
            pl.BlockSpec((2, hid_c), lambda i: (0, 0)),
            pl.BlockSpec((2, hid_c), lambda i: (0, 0)),
            pl.BlockSpec((2, hid_c, out_c), lambda i: (0, 0, 0)),
        ],
        out_specs=pl.BlockSpec((bn_rows, 2 * out_c), lambda i: (i, 0)),
        out_shape=jax.ShapeDtypeStruct((npad, 2 * out_c), F32),
    )(agg2, stats2, gam1, bet1, w2)

    p3 = _make_agg_kernel(npad, nch, 2 * out_c)(h3cat, row, col, norm)

    y = pl.pallas_call(
        functools.partial(_final_body, out_c),
        grid=(nb,),
        in_specs=[
            pl.BlockSpec((NC, bn_rows, 2 * out_c), lambda i: (0, i, 0)),
            pl.BlockSpec((bn_rows, 2 * out_c), lambda i: (i, 0)),
            pl.BlockSpec((bn_rows, 1), lambda i: (i, 0)),
            pl.BlockSpec((2, out_c), lambda i: (0, 0)),
        ],
        out_specs=pl.BlockSpec((2, bn_rows, out_c), lambda i: (0, i, 0)),
        out_shape=jax.ShapeDtypeStruct((2, npad, out_c), F32),
    )(p3, h3cat, dinv2, b2)

    return y[:, :n]
